# Initial kernel scaffold; baseline (speedup 1.0000x reference)
import jax
import jax.numpy as jnp
from jax.experimental import pallas as pl


def kernel(x, table, positional_tokens):
    raise NotImplementedError("write your pallas kernel here")


# speedup vs baseline: 1.0550x; 1.0550x over previous
import functools

import jax
import jax.numpy as jnp
from jax import lax
from jax.experimental import pallas as pl
from jax.experimental.pallas import tpu as pltpu

B = 32
C = 3
NP = 16
PATCH = 32
EMB = PATCH * PATCH * C
NUM_EMB = NP * NP
ROWS_C = PATCH * PATCH


def _sc_gather(table, idx):
    from jax.experimental.pallas import tpu_sc as plsc

    mesh = plsc.VectorSubcoreMesh(core_axis_name="c", subcore_axis_name="s")
    nw = 32
    bpw = NUM_EMB // nw

    @functools.partial(
        pl.kernel,
        mesh=mesh,
        out_type=jax.ShapeDtypeStruct((NUM_EMB, EMB), jnp.float32),
        scratch_types=[
            pltpu.VMEM((bpw,), jnp.int32),
            pltpu.VMEM((bpw, EMB), jnp.float32),
            pltpu.SemaphoreType.DMA,
        ],
    )
    def k(table_hbm, idx_hbm, out_hbm, idx_v, rows_v, sem):
        wid = lax.axis_index("s") * 2 + lax.axis_index("c")
        base = wid * bpw
        pltpu.sync_copy(idx_hbm.at[pl.ds(base, bpw)], idx_v)
        pltpu.async_copy(table_hbm.at[idx_v], rows_v, sem).wait()
        pltpu.sync_copy(rows_v, out_hbm.at[pl.ds(base, bpw)])

    return k(table, idx)


def _tc_body(x_ref, pe_ref, o_ref):
    xb = x_ref[0, 0]
    x4 = xb.reshape(NP, PATCH, NP, PATCH)
    p = jnp.transpose(x4, (1, 3, 0, 2))
    o_ref[0] = p.reshape(ROWS_C, NUM_EMB) + pe_ref[...]


def _tc_unfold_add(x, pe_r, interpret=False):
    return pl.pallas_call(
        _tc_body,
        grid=(C, B),
        in_specs=[
            pl.BlockSpec((1, 1, NP * PATCH, NP * PATCH), lambda c, b: (b, c, 0, 0)),
            pl.BlockSpec((ROWS_C, NUM_EMB), lambda c, b: (c, 0)),
        ],
        out_specs=pl.BlockSpec((1, ROWS_C, NUM_EMB), lambda c, b: (b, c, 0)),
        out_shape=jax.ShapeDtypeStruct((B, C * ROWS_C, NUM_EMB), jnp.float32),
        compiler_params=pltpu.CompilerParams(
            dimension_semantics=("parallel", "parallel"),
        ),
        interpret=interpret,
    )(x, pe_r)


def kernel(x, table, positional_tokens):
    pe = _sc_gather(table, positional_tokens.reshape(NUM_EMB))
    pe_r = pe.reshape(C * ROWS_C, NUM_EMB)
    out = _tc_unfold_add(x, pe_r)
    return out.reshape(B, NUM_EMB, EMB)


# speedup vs baseline: 2.0033x; 1.8990x over previous
import functools

import jax
import jax.numpy as jnp
from jax import lax
from jax.experimental import pallas as pl
from jax.experimental.pallas import tpu as pltpu

B = 32
C = 3
NP = 16
PATCH = 32
EMB = PATCH * PATCH * C
NUM_EMB = NP * NP
ROWS_C = PATCH * PATCH


def _sc_gather(table, idx):
    from jax.experimental.pallas import tpu_sc as plsc

    mesh = plsc.VectorSubcoreMesh(core_axis_name="c", subcore_axis_name="s")
    nw = 32
    bpw = NUM_EMB // nw

    @functools.partial(
        pl.kernel,
        mesh=mesh,
        out_type=jax.ShapeDtypeStruct((NUM_EMB, EMB), jnp.float32),
        scratch_types=[
            pltpu.VMEM((bpw,), jnp.int32),
            pltpu.VMEM((bpw, EMB), jnp.float32),
            pltpu.SemaphoreType.DMA,
        ],
    )
    def k(table_hbm, idx_hbm, out_hbm, idx_v, rows_v, sem):
        wid = lax.axis_index("s") * 2 + lax.axis_index("c")
        base = wid * bpw
        pltpu.sync_copy(idx_hbm.at[pl.ds(base, bpw)], idx_v)
        pltpu.async_copy(table_hbm.at[idx_v], rows_v, sem).wait()
        pltpu.sync_copy(rows_v, out_hbm.at[pl.ds(base, bpw)])

    return k(table, idx)


def _tc_body(x_ref, pe_ref, o_ref):
    xb = x_ref[0, 0]
    a = xb.reshape(NP, PATCH, 512).transpose(0, 2, 1)
    b = lax.bitcast_convert_type(a.reshape(NP, NP, PATCH, PATCH), jnp.int32)
    d2 = lax.bitcast_convert_type(b.reshape(NUM_EMB, ROWS_C), jnp.float32)
    p0 = d2.T
    p = p0.reshape(PATCH, PATCH, NUM_EMB).transpose(1, 0, 2).reshape(ROWS_C, NUM_EMB)
    o_ref[0] = p + pe_ref[...]


def _tc_unfold_add(x, pe_r, interpret=False):
    return pl.pallas_call(
        _tc_body,
        grid=(C, B),
        in_specs=[
            pl.BlockSpec((1, 1, NP * PATCH, NP * PATCH), lambda c, b: (b, c, 0, 0)),
            pl.BlockSpec((ROWS_C, NUM_EMB), lambda c, b: (c, 0)),
        ],
        out_specs=pl.BlockSpec((1, ROWS_C, NUM_EMB), lambda c, b: (b, c, 0)),
        out_shape=jax.ShapeDtypeStruct((B, C * ROWS_C, NUM_EMB), jnp.float32),
        compiler_params=pltpu.CompilerParams(
            dimension_semantics=("parallel", "parallel"),
        ),
        interpret=interpret,
    )(x, pe_r)


def kernel(x, table, positional_tokens):
    pe = _sc_gather(table, positional_tokens.reshape(NUM_EMB))
    pe_r = pe.reshape(C * ROWS_C, NUM_EMB)
    out = _tc_unfold_add(x, pe_r)
    return out.reshape(B, NUM_EMB, EMB)


# speedup vs baseline: 2.7317x; 1.3636x over previous
import functools

import jax
import jax.numpy as jnp
from jax import lax
from jax.experimental import pallas as pl
from jax.experimental.pallas import tpu as pltpu

B = 32
C = 3
NP = 16
PATCH = 32
EMB = PATCH * PATCH * C
NUM_EMB = NP * NP
ROWS_C = PATCH * PATCH


def _sc_gather(table, idx):
    from jax.experimental.pallas import tpu_sc as plsc

    mesh = plsc.VectorSubcoreMesh(core_axis_name="c", subcore_axis_name="s")
    nw = 32
    bpw = NUM_EMB // nw

    @functools.partial(
        pl.kernel,
        mesh=mesh,
        out_type=jax.ShapeDtypeStruct((NUM_EMB, EMB), jnp.float32),
        scratch_types=[
            pltpu.VMEM((bpw,), jnp.int32),
            pltpu.VMEM((bpw, EMB), jnp.float32),
            pltpu.SemaphoreType.DMA,
        ],
    )
    def k(table_hbm, idx_hbm, out_hbm, idx_v, rows_v, sem):
        wid = lax.axis_index("s") * 2 + lax.axis_index("c")
        base = wid * bpw
        pltpu.sync_copy(idx_hbm.at[pl.ds(base, bpw)], idx_v)
        pltpu.async_copy(table_hbm.at[idx_v], rows_v, sem).wait()
        pltpu.sync_copy(rows_v, out_hbm.at[pl.ds(base, bpw)])

    return k(table, idx)


def _unfold_channel(xb):
    a = xb.reshape(NP, PATCH, 512).transpose(0, 2, 1)
    b4 = lax.bitcast_convert_type(a.reshape(NP, NP, PATCH, PATCH), jnp.int32)
    d2 = lax.bitcast_convert_type(b4.reshape(NUM_EMB, ROWS_C), jnp.float32)
    p0 = d2.T
    return p0.reshape(PATCH, PATCH, NUM_EMB).transpose(1, 0, 2).reshape(ROWS_C, NUM_EMB)


def _tc_body(x_ref, pe_ref, o_ref):
    ps = [_unfold_channel(x_ref[0, c]) for c in range(C)]
    pf = jnp.concatenate(ps, axis=0) + pe_ref[...]
    g = lax.bitcast_convert_type(pf.reshape(NUM_EMB, C * 4, NUM_EMB), jnp.int32)
    o_ref[0] = lax.bitcast_convert_type(g.reshape(NUM_EMB, EMB), jnp.float32)


def _tc_unfold_add(x, pe_r, interpret=False):
    return pl.pallas_call(
        _tc_body,
        grid=(B,),
        in_specs=[
            pl.BlockSpec((1, C, NP * PATCH, NP * PATCH), lambda b: (b, 0, 0, 0)),
            pl.BlockSpec((C * ROWS_C, NUM_EMB), lambda b: (0, 0)),
        ],
        out_specs=pl.BlockSpec((1, NUM_EMB, EMB), lambda b: (b, 0, 0)),
        out_shape=jax.ShapeDtypeStruct((B, NUM_EMB, EMB), jnp.float32),
        compiler_params=pltpu.CompilerParams(
            dimension_semantics=("parallel",),
        ),
        interpret=interpret,
    )(x, pe_r)


def kernel(x, table, positional_tokens):
    pe = _sc_gather(table, positional_tokens.reshape(NUM_EMB))
    pe_r = pe.reshape(C * ROWS_C, NUM_EMB)
    return _tc_unfold_add(x, pe_r)


# speedup vs baseline: 2.7978x; 1.0242x over previous
import functools

import jax
import jax.numpy as jnp
from jax import lax
from jax.experimental import pallas as pl
from jax.experimental.pallas import tpu as pltpu

B = 32
C = 3
NP = 16
PATCH = 32
EMB = PATCH * PATCH * C
NUM_EMB = NP * NP
ROWS_C = PATCH * PATCH


def _sc_gather(table, idx):
    from jax.experimental.pallas import tpu_sc as plsc

    mesh = plsc.VectorSubcoreMesh(core_axis_name="c", subcore_axis_name="s")
    nw = 32
    bpw = NUM_EMB // nw

    @functools.partial(
        pl.kernel,
        mesh=mesh,
        out_type=jax.ShapeDtypeStruct((NUM_EMB, EMB), jnp.float32),
        scratch_types=[
            pltpu.VMEM((bpw,), jnp.int32),
            pltpu.VMEM((bpw, EMB), jnp.float32),
            pltpu.SemaphoreType.DMA,
        ],
    )
    def k(table_hbm, idx_hbm, out_hbm, idx_v, rows_v, sem):
        wid = lax.axis_index("s") * 2 + lax.axis_index("c")
        base = wid * bpw
        pltpu.sync_copy(idx_hbm.at[pl.ds(base, bpw)], idx_v)
        pltpu.async_copy(table_hbm.at[idx_v], rows_v, sem).wait()
        pltpu.sync_copy(rows_v, out_hbm.at[pl.ds(base, bpw)])

    return k(table, idx)


def _unfold_channel(xb):
    a = xb.reshape(NP, PATCH, 512).transpose(0, 2, 1)
    b4 = lax.bitcast_convert_type(a.reshape(NP, NP, PATCH, PATCH), jnp.int32)
    d2 = lax.bitcast_convert_type(b4.reshape(NUM_EMB, ROWS_C), jnp.float32)
    p0 = d2.T
    return p0.reshape(PATCH, PATCH, NUM_EMB).transpose(1, 0, 2).reshape(ROWS_C, NUM_EMB)


def _tc_body(x_ref, pe_ref, o_ref):
    ps = [_unfold_channel(x_ref[0, c]) for c in range(C)]
    pf = jnp.concatenate(ps, axis=0)
    g = lax.bitcast_convert_type(pf.reshape(NUM_EMB, C * 4, NUM_EMB), jnp.int32)
    o_ref[0] = lax.bitcast_convert_type(g.reshape(NUM_EMB, EMB), jnp.float32) + pe_ref[...]


def _tc_unfold_add(x, pe_r, interpret=False):
    return pl.pallas_call(
        _tc_body,
        grid=(B,),
        in_specs=[
            pl.BlockSpec((1, C, NP * PATCH, NP * PATCH), lambda b: (b, 0, 0, 0)),
            pl.BlockSpec((NUM_EMB, EMB), lambda b: (0, 0)),
        ],
        out_specs=pl.BlockSpec((1, NUM_EMB, EMB), lambda b: (b, 0, 0)),
        out_shape=jax.ShapeDtypeStruct((B, NUM_EMB, EMB), jnp.float32),
        compiler_params=pltpu.CompilerParams(
            dimension_semantics=("parallel",),
        ),
        interpret=interpret,
    )(x, pe_r)


def kernel(x, table, positional_tokens):
    pe = _sc_gather(table, positional_tokens.reshape(NUM_EMB))
    return _tc_unfold_add(x, pe)


# speedup vs baseline: 4.8313x; 1.7268x over previous
import functools

import jax
import jax.numpy as jnp
from jax import lax
from jax.experimental import pallas as pl
from jax.experimental.pallas import tpu as pltpu

B = 32
C = 3
NP = 16
PATCH = 32
EMB = PATCH * PATCH * C
NUM_EMB = NP * NP
ROWS_C = PATCH * PATCH


def _sc_gather(table, idx):
    from jax.experimental.pallas import tpu_sc as plsc

    mesh = plsc.VectorSubcoreMesh(core_axis_name="c", subcore_axis_name="s")
    nw = 32
    bpw = NUM_EMB // nw

    @functools.partial(
        pl.kernel,
        mesh=mesh,
        out_type=jax.ShapeDtypeStruct((NUM_EMB, EMB), jnp.float32),
        scratch_types=[
            pltpu.VMEM((bpw,), jnp.int32),
            pltpu.VMEM((bpw, EMB), jnp.float32),
            pltpu.SemaphoreType.DMA,
        ],
    )
    def k(table_hbm, idx_hbm, out_hbm, idx_v, rows_v, sem):
        wid = lax.axis_index("s") * 2 + lax.axis_index("c")
        base = wid * bpw
        pltpu.sync_copy(idx_hbm.at[pl.ds(base, bpw)], idx_v)
        pltpu.async_copy(table_hbm.at[idx_v], rows_v, sem).wait()
        pltpu.sync_copy(rows_v, out_hbm.at[pl.ds(base, bpw)])

    return k(table, idx)


def _tc_body(x_ref, pe_ref, o_ref):
    _bc = lax.bitcast_convert_type
    xb = x_ref[0]
    y = xb.reshape(C, NP, PATCH, 512).transpose(0, 2, 1, 3)
    y2 = _bc(y.reshape(C, PATCH, NP * 512), jnp.int32)
    i2 = _bc(y2.reshape(C * PATCH, NP * 512), jnp.float32)
    h = i2.T
    h = h.reshape(NUM_EMB, PATCH, C * PATCH).transpose(1, 0, 2)
    f3 = h.reshape(NP * 512, C * PATCH).T
    g = _bc(f3.reshape(C * PATCH, PATCH, NUM_EMB), jnp.int32)
    pf = g.reshape(C * ROWS_C, NUM_EMB)
    pf = _bc(_bc(pf, jnp.float32), jnp.int32)
    gg = pf.reshape(NUM_EMB, C * 4, NUM_EMB)
    o_ref[0] = _bc(gg.reshape(NUM_EMB, EMB), jnp.float32) + pe_ref[...]


def _tc_unfold_add(x, pe_r, interpret=False):
    return pl.pallas_call(
        _tc_body,
        grid=(B,),
        in_specs=[
            pl.BlockSpec((1, C, NP * PATCH, NP * PATCH), lambda b: (b, 0, 0, 0)),
            pl.BlockSpec((NUM_EMB, EMB), lambda b: (0, 0)),
        ],
        out_specs=pl.BlockSpec((1, NUM_EMB, EMB), lambda b: (b, 0, 0)),
        out_shape=jax.ShapeDtypeStruct((B, NUM_EMB, EMB), jnp.float32),
        compiler_params=pltpu.CompilerParams(
            dimension_semantics=("parallel",),
        ),
        interpret=interpret,
    )(x, pe_r)


def kernel(x, table, positional_tokens):
    pe = _sc_gather(table, positional_tokens.reshape(NUM_EMB))
    return _tc_unfold_add(x, pe)


# speedup vs baseline: 4.8365x; 1.0011x over previous
import functools

import jax
import jax.numpy as jnp
from jax import lax
from jax.experimental import pallas as pl
from jax.experimental.pallas import tpu as pltpu

B = 32
C = 3
NP = 16
PATCH = 32
EMB = PATCH * PATCH * C
NUM_EMB = NP * NP
ROWS_C = PATCH * PATCH


def _sc_gather(table, idx):
    from jax.experimental.pallas import tpu_sc as plsc

    mesh = plsc.VectorSubcoreMesh(core_axis_name="c", subcore_axis_name="s")
    nw = 32
    bpw = NUM_EMB // nw

    @functools.partial(
        pl.kernel,
        mesh=mesh,
        out_type=jax.ShapeDtypeStruct((NUM_EMB, EMB), jnp.float32),
        scratch_types=[
            pltpu.VMEM((bpw,), jnp.int32),
            pltpu.VMEM((bpw, EMB), jnp.float32),
            pltpu.SemaphoreType.DMA,
        ],
    )
    def k(table_hbm, idx_hbm, out_hbm, idx_v, rows_v, sem):
        wid = lax.axis_index("s") * 2 + lax.axis_index("c")
        base = wid * bpw
        pltpu.sync_copy(idx_hbm.at[0, pl.ds(base, bpw)], idx_v)
        pltpu.async_copy(table_hbm.at[idx_v], rows_v, sem).wait()
        pltpu.sync_copy(rows_v, out_hbm.at[pl.ds(base, bpw)])

    return k(table, idx)


def _tc_body(x_ref, pe_ref, o_ref):
    _bc = lax.bitcast_convert_type
    xb = x_ref[0]
    y = xb.reshape(C, NP, PATCH, 512).transpose(0, 2, 1, 3)
    y2 = _bc(y.reshape(C, PATCH, NP * 512), jnp.int32)
    i2 = _bc(y2.reshape(C * PATCH, NP * 512), jnp.float32)
    h = i2.T
    h = h.reshape(NUM_EMB, PATCH, C * PATCH).transpose(1, 0, 2)
    f3 = h.reshape(NP * 512, C * PATCH).T
    g = _bc(f3.reshape(C * PATCH, PATCH, NUM_EMB), jnp.int32)
    pf = g.reshape(C * ROWS_C, NUM_EMB)
    pf = _bc(_bc(pf, jnp.float32), jnp.int32)
    gg = pf.reshape(NUM_EMB, C * 4, NUM_EMB)
    o_ref[0] = _bc(gg.reshape(NUM_EMB, EMB), jnp.float32) + pe_ref[...]


def _tc_unfold_add(x, pe_r, interpret=False):
    return pl.pallas_call(
        _tc_body,
        grid=(B,),
        in_specs=[
            pl.BlockSpec((1, C, NP * PATCH, NP * PATCH), lambda b: (b, 0, 0, 0)),
            pl.BlockSpec((NUM_EMB, EMB), lambda b: (0, 0)),
        ],
        out_specs=pl.BlockSpec((1, NUM_EMB, EMB), lambda b: (b, 0, 0)),
        out_shape=jax.ShapeDtypeStruct((B, NUM_EMB, EMB), jnp.float32),
        compiler_params=pltpu.CompilerParams(
            dimension_semantics=("parallel",),
        ),
        interpret=interpret,
    )(x, pe_r)


def kernel(x, table, positional_tokens):
    pe = _sc_gather(table, positional_tokens)
    return _tc_unfold_add(x, pe)
